# Initial kernel scaffold; baseline (speedup 1.0000x reference)
import jax
import jax.numpy as jnp
from jax.experimental import pallas as pl


def kernel(x, subjectID, W_net, b_net, W_cal, b_cal):
    raise NotImplementedError("write your pallas kernel here")


# speedup vs baseline: 4.7664x; 4.7664x over previous
import functools

import jax
import jax.numpy as jnp
from jax import lax
from jax.experimental import pallas as pl
from jax.experimental.pallas import tpu as pltpu
from jax.experimental.pallas import tpu_sc as plsc

N = 16384
D = 128
S = 64
PD = 16

_NC = 2
_NS = 16
_NW = _NC * _NS
_BPW = N // _NW
_CH = 128
_NCH = _BPW // _CH

_sc_mesh = plsc.VectorSubcoreMesh(core_axis_name="c", subcore_axis_name="s")


@functools.partial(
    pl.kernel,
    out_type=jax.ShapeDtypeStruct((N, PD), jnp.float32),
    mesh=_sc_mesh,
    scratch_types=[
        pltpu.VMEM((_NCH, _CH), jnp.int32),
        pltpu.VMEM((_CH, PD), jnp.float32),
        pltpu.SemaphoreType.DMA,
    ],
    compiler_params=pltpu.CompilerParams(use_tc_tiling_on_sc=False),
)
def _sc_gather(p_hbm, sid_hbm, out_hbm, idx_v, rows_v, sem):
    wid = lax.axis_index("s") * _NC + lax.axis_index("c")
    pltpu.sync_copy(sid_hbm.at[pl.ds(wid * _NCH, _NCH)], idx_v)
    for i in range(_NCH):
        pltpu.async_copy(p_hbm.at[idx_v.at[i]], rows_v, sem).wait()
        pltpu.sync_copy(rows_v, out_hbm.at[pl.ds(wid * _BPW + i * _CH, _CH)])


_BT = 2048


def _tc_body(x_ref, g_ref, w_ref, b_ref, o_ref):
    y = jnp.dot(x_ref[...], w_ref[...], preferred_element_type=jnp.float32)
    y = y + b_ref[...]
    g = g_ref[...]
    o_ref[...] = y[:, 0:1] * g[:, 0:2] + y[:, 1:2] * g[:, 2:4] + g[:, 4:6]


def kernel(x, subjectID, W_net, b_net, W_cal, b_cal):
    p = jnp.zeros((S, PD), jnp.float32)
    p = p.at[:, 0:4].set(W_cal.reshape(S, 4))
    p = p.at[:, 4:6].set(b_cal)
    sid = subjectID.astype(jnp.int32).reshape(N // _CH, _CH)
    g = _sc_gather(p, sid)
    out = pl.pallas_call(
        _tc_body,
        grid=(N // _BT,),
        in_specs=[
            pl.BlockSpec((_BT, D), lambda i: (i, 0)),
            pl.BlockSpec((_BT, PD), lambda i: (i, 0)),
            pl.BlockSpec((D, 2), lambda i: (0, 0)),
            pl.BlockSpec((1, 2), lambda i: (0, 0)),
        ],
        out_specs=pl.BlockSpec((_BT, 2), lambda i: (i, 0)),
        out_shape=jax.ShapeDtypeStruct((N, 2), jnp.float32),
    )(x, g, W_net, b_net.reshape(1, 2))
    return out


# speedup vs baseline: 5.7820x; 1.2131x over previous
import functools

import jax
import jax.numpy as jnp
import numpy as np
from jax import lax
from jax.experimental import pallas as pl
from jax.experimental.pallas import tpu as pltpu
from jax.experimental.pallas import tpu_sc as plsc

N = 16384
D = 128
NSUBJ = 64
PD = 16

_NC = 2
_NS = 16
_NW = _NC * _NS
_BPW = N // _NW
_L = 16

_sc_mesh = plsc.VectorSubcoreMesh(core_axis_name="c", subcore_axis_name="s")


@functools.partial(
    pl.kernel,
    out_type=jax.ShapeDtypeStruct((N, PD), jnp.float32),
    mesh=_sc_mesh,
    scratch_types=[
        pltpu.VMEM((NSUBJ, 4), jnp.float32),
        pltpu.VMEM((NSUBJ, 2), jnp.float32),
        pltpu.VMEM((_BPW,), jnp.int32),
        pltpu.VMEM((_BPW, PD), jnp.float32),
        pltpu.SemaphoreType.DMA,
    ],
    compiler_params=pltpu.CompilerParams(
        use_tc_tiling_on_sc=False, needs_layout_passes=False
    ),
)
def _sc_gather(w_hbm, b_hbm, sid_hbm, out_hbm, w_v, b_v, idx_v, out_v, sem):
    wid = lax.axis_index("s") * _NC + lax.axis_index("c")
    pltpu.sync_copy(w_hbm, w_v)
    pltpu.sync_copy(b_hbm, b_v)
    pltpu.sync_copy(sid_hbm.at[pl.ds(wid * _BPW, _BPW)], idx_v)
    lanes = lax.iota(jnp.int32, _L)
    cols = [jnp.full((_L,), j, jnp.int32) for j in range(6)]
    for i in range(_BPW // _L):
        sidv = idx_v[pl.ds(i * _L, _L)]
        rows = lanes + (i * _L)
        for j in range(4):
            vals = plsc.load_gather(w_v, [sidv, cols[j]])
            plsc.store_scatter(out_v, [rows, cols[j]], vals)
        for j in range(2):
            vals = plsc.load_gather(b_v, [sidv, cols[j]])
            plsc.store_scatter(out_v, [rows, cols[4 + j]], vals)
    pltpu.sync_copy(out_v, out_hbm.at[pl.ds(wid * _BPW, _BPW)])


_BT = 2048


def _selectors():
    r2 = lax.broadcasted_iota(jnp.int32, (2, PD), 0)
    c2 = lax.broadcasted_iota(jnp.int32, (2, PD), 1)
    s_sel = (c2 // 2 == r2).astype(jnp.float32)
    c1 = lax.broadcasted_iota(jnp.int32, (1, PD), 1)
    c_sel = (c1 // 2 == 2).astype(jnp.float32)
    rj = lax.broadcasted_iota(jnp.int32, (PD, 2), 0)
    ck = lax.broadcasted_iota(jnp.int32, (PD, 2), 1)
    r_sel = ((rj % 2 == ck) & (rj < 6)).astype(jnp.float32)
    return s_sel, c_sel, r_sel


def _tc_body(x_ref, g_ref, w_ref, b_ref, o_ref):
    s_sel, c_sel, r_sel = _selectors()
    y = jnp.dot(x_ref[...], w_ref[...], preferred_element_type=jnp.float32)
    y = y + b_ref[...]
    ybig = jnp.dot(y, s_sel, preferred_element_type=jnp.float32) + c_sel
    t = g_ref[...] * ybig
    o_ref[...] = jnp.dot(t, r_sel, preferred_element_type=jnp.float32)


def kernel(x, subjectID, W_net, b_net, W_cal, b_cal):
    g = _sc_gather(W_cal.reshape(NSUBJ, 4), b_cal, subjectID.astype(jnp.int32))
    out = pl.pallas_call(
        _tc_body,
        grid=(N // _BT,),
        in_specs=[
            pl.BlockSpec((_BT, D), lambda i: (i, 0)),
            pl.BlockSpec((_BT, PD), lambda i: (i, 0)),
            pl.BlockSpec((D, 2), lambda i: (0, 0)),
            pl.BlockSpec((1, 2), lambda i: (0, 0)),
        ],
        out_specs=pl.BlockSpec((_BT, 2), lambda i: (i, 0)),
        out_shape=jax.ShapeDtypeStruct((N, 2), jnp.float32),
    )(x, g, W_net, b_net.reshape(1, 2))
    return out
